# baseline (speedup 1.0000x reference)
import functools

import jax
import jax.numpy as jnp
from jax import lax
from jax.experimental import pallas as pl
from jax.experimental.pallas import tpu as pltpu
from jax.experimental.pallas import tpu_sc as plsc

B, T, H, E, F, K = 1, 2048, 768, 8, 3072, 2
P = T * K
TM = 128
NT = P // TM + E
S = NT * TM

NC, NS = 2, 16
NW = NC * NS
RG = S // NW
RC = T // NW

_NEG = -1e30


def _router_body(x_ref, rw_ref, st_ref, sw_ref, pos_ref, te_ref, aux_ref):
    x = x_ref[...]
    rw = rw_ref[...]
    logits = lax.dot_general(x, rw, (((1,), (1,)), ((), ())),
                             preferred_element_type=jnp.float32)
    ids8 = lax.broadcasted_iota(jnp.int32, (T, E), 1)
    v0 = jnp.max(logits, axis=1, keepdims=True)
    i0 = jnp.min(jnp.where(logits == v0, ids8, E), axis=1, keepdims=True)
    masked = jnp.where(ids8 == i0, _NEG, logits)
    v1 = jnp.max(masked, axis=1, keepdims=True)
    i1 = jnp.min(jnp.where(masked == v1, ids8, E), axis=1, keepdims=True)
    e1 = jnp.exp(v1 - v0)
    w0 = 1.0 / (1.0 + e1)
    w1 = 1.0 - w0

    oh0 = (ids8 == i0).astype(jnp.float32)
    oh1 = (ids8 == i1).astype(jnp.float32)
    fullcnt = oh0 + oh1

    ri = lax.broadcasted_iota(jnp.int32, (TM, TM), 0)
    ci = lax.broadcasted_iota(jnp.int32, (TM, TM), 1)
    lincl = (ri >= ci).astype(jnp.float32)
    nb = T // TM
    acc = jnp.zeros((1, E), jnp.float32)
    cparts = []
    for b in range(nb):
        blk = fullcnt[b * TM:(b + 1) * TM, :]
        incl = lax.dot(lincl, blk, preferred_element_type=jnp.float32)
        cparts.append(incl - blk + acc)
        acc = acc + incl[TM - 1:TM, :]
    C = jnp.concatenate(cparts, axis=0)
    counts = acc

    tiles = jnp.floor((counts + (TM - 1)) * (1.0 / TM))
    er = lax.broadcasted_iota(jnp.int32, (E, E), 0)
    ec = lax.broadcasted_iota(jnp.int32, (E, E), 1)
    uexcl = (er < ec).astype(jnp.float32)
    ptile_excl = lax.dot(tiles, uexcl, preferred_element_type=jnp.float32)
    ptile_end = ptile_excl + tiles
    pstart = ptile_excl * TM

    pos0 = jnp.sum(oh0 * (pstart + C), axis=1, keepdims=True)
    pos1 = jnp.sum(oh1 * (pstart + C + oh0), axis=1, keepdims=True)
    pos_ref[...] = jnp.concatenate([pos0, pos1], axis=1).astype(jnp.int32)

    ti = lax.broadcasted_iota(jnp.int32, (NT, E), 0).astype(jnp.float32)
    te = jnp.sum((ptile_end <= ti).astype(jnp.int32), axis=1, keepdims=True)
    te_ref[...] = jnp.minimum(te, E - 1)

    tv = lax.broadcasted_iota(jnp.int32, (T, 1), 0).astype(jnp.float32)
    srow0 = lax.broadcasted_iota(jnp.int32, (1, TM), 1).astype(jnp.float32)
    for s in range(NT):
        srow = srow0 + (s * TM)
        eq0 = pos0 == srow
        eq1 = pos1 == srow
        tok = (jnp.sum(jnp.where(eq0, tv, 0.0), axis=0, keepdims=True)
               + jnp.sum(jnp.where(eq1, tv, 0.0), axis=0, keepdims=True))
        wgt = (jnp.sum(jnp.where(eq0, w0, 0.0), axis=0, keepdims=True)
               + jnp.sum(jnp.where(eq1, w1, 0.0), axis=0, keepdims=True))
        st_ref[s:s + 1, :] = tok.astype(jnp.int32)
        sw_ref[s:s + 1, :] = wgt

    p = jnp.exp(logits - v0)
    probs = p / jnp.sum(p, axis=1, keepdims=True)
    Pm = jnp.sum(probs, axis=0, keepdims=True) * (1.0 / T)
    f = counts * (1.0 / P)
    aux_ref[0, 0] = E * jnp.sum(f * Pm)


def _router(x, router_w):
    return pl.pallas_call(
        _router_body,
        out_shape=(
            jax.ShapeDtypeStruct((NT, TM), jnp.int32),
            jax.ShapeDtypeStruct((NT, TM), jnp.float32),
            jax.ShapeDtypeStruct((T, K), jnp.int32),
            jax.ShapeDtypeStruct((NT, 1), jnp.int32),
            jax.ShapeDtypeStruct((1, 1), jnp.float32),
        ),
        in_specs=[
            pl.BlockSpec((T, H), lambda: (0, 0)),
            pl.BlockSpec((E, H), lambda: (0, 0)),
        ],
        out_specs=(
            pl.BlockSpec((NT, TM), lambda: (0, 0)),
            pl.BlockSpec((NT, TM), lambda: (0, 0)),
            pl.BlockSpec((T, K), lambda: (0, 0)),
            pl.BlockSpec((NT, 1), lambda: (0, 0)),
            pl.BlockSpec(memory_space=pltpu.SMEM),
        ),
    )(x, router_w)


def _sc_mesh():
    return plsc.VectorSubcoreMesh(core_axis_name="c", subcore_axis_name="s",
                                  num_cores=NC, num_subcores=NS)


_CH = 80


def _gather_body(st_hbm, x_hbm, xs_hbm, idx_a, idx_b, rows_a, rows_b,
                 gsem, ssem):
    wid = lax.axis_index("s") * NC + lax.axis_index("c")
    base = wid * RG
    pltpu.sync_copy(st_hbm.at[pl.ds(base, _CH)], idx_a)
    pltpu.sync_copy(st_hbm.at[pl.ds(base + _CH, _CH)], idx_b)
    ga = pltpu.async_copy(x_hbm.at[idx_a], rows_a, gsem)
    gb = pltpu.async_copy(x_hbm.at[idx_b], rows_b, gsem)
    ga.wait()
    sa = pltpu.async_copy(rows_a, xs_hbm.at[pl.ds(base, _CH)], ssem)
    gb.wait()
    sb = pltpu.async_copy(rows_b, xs_hbm.at[pl.ds(base + _CH, _CH)], ssem)
    sa.wait()
    sb.wait()


def _sc_gather(x, st_flat):
    k = pl.kernel(
        _gather_body,
        out_type=jax.ShapeDtypeStruct((S, H), jnp.float32),
        mesh=_sc_mesh(),
        scratch_types=[
            pltpu.VMEM((_CH,), jnp.int32),
            pltpu.VMEM((_CH,), jnp.int32),
            pltpu.VMEM((_CH, H), jnp.float32),
            pltpu.VMEM((_CH, H), jnp.float32),
            pltpu.SemaphoreType.DMA,
            pltpu.SemaphoreType.DMA,
        ],
    )
    return k(st_flat, x)


def _gelu(x):
    return x * 0.5 * (1.0 + lax.erf(x * 0.7071067811865476))


def _ffn_body(te_ref, xs_ref, w1_ref, b1_ref, w2_ref, b2_ref, sw_ref,
              out_ref, w1s, w2s):
    i = pl.program_id(0)
    e = te_ref[i]
    eprev = te_ref[jnp.maximum(i - 1, 0)]

    @pl.when((i == 0) | (e != eprev))
    def _cast_weights():
        w1s[...] = w1_ref[0].astype(jnp.bfloat16)
        w2s[...] = w2_ref[0].astype(jnp.bfloat16)

    xb = xs_ref[...].astype(jnp.bfloat16)
    h = lax.dot(xb, w1s[...], preferred_element_type=jnp.float32)
    h = _gelu(h + b1_ref[0])
    y = lax.dot(h.astype(jnp.bfloat16), w2s[...],
                preferred_element_type=jnp.float32) + b2_ref[0]
    ri = lax.broadcasted_iota(jnp.int32, (TM, TM), 0)
    ci = lax.broadcasted_iota(jnp.int32, (TM, TM), 1)
    wcol = jnp.sum(jnp.where(ri == ci, sw_ref[0], 0.0), axis=1,
                   keepdims=True)
    out_ref[...] = wcol * y


def _ffn(xs, w1, b1, w2, b2, sw, te):
    grid_spec = pltpu.PrefetchScalarGridSpec(
        num_scalar_prefetch=1,
        grid=(NT,),
        in_specs=[
            pl.BlockSpec((TM, H), lambda i, te: (i, 0)),
            pl.BlockSpec((1, H, F), lambda i, te: (te[i], 0, 0)),
            pl.BlockSpec((1, 1, F), lambda i, te: (te[i], 0, 0)),
            pl.BlockSpec((1, F, H), lambda i, te: (te[i], 0, 0)),
            pl.BlockSpec((1, 1, H), lambda i, te: (te[i], 0, 0)),
            pl.BlockSpec((1, 1, TM), lambda i, te: (i, 0, 0)),
        ],
        out_specs=pl.BlockSpec((TM, H), lambda i, te: (i, 0)),
        scratch_shapes=[
            pltpu.VMEM((H, F), jnp.bfloat16),
            pltpu.VMEM((F, H), jnp.bfloat16),
        ],
    )
    return pl.pallas_call(
        _ffn_body,
        grid_spec=grid_spec,
        out_shape=jax.ShapeDtypeStruct((S, H), jnp.float32),
    )(te, xs, w1, b1.reshape(E, 1, F), w2, b2.reshape(E, 1, H),
      sw.reshape(NT, 1, TM))


def _combine_body(p0_hbm, p1_hbm, ys_hbm, out_hbm, idx0, idx1, rows0, rows1,
                  sem):
    wid = lax.axis_index("s") * NC + lax.axis_index("c")
    base = wid * RC
    pltpu.sync_copy(p0_hbm.at[pl.ds(base, RC)], idx0)
    pltpu.sync_copy(p1_hbm.at[pl.ds(base, RC)], idx1)
    c0 = pltpu.async_copy(ys_hbm.at[idx0], rows0, sem)
    c1 = pltpu.async_copy(ys_hbm.at[idx1], rows1, sem)
    c0.wait()
    c1.wait()

    def _row(j, _):
        def _chunk(i, _):
            sl = pl.ds(i * 16, 16)
            rows0[j, sl] = rows0[j, sl] + rows1[j, sl]
            return 0
        return lax.fori_loop(0, H // 16, _chunk, 0, unroll=4)

    lax.fori_loop(0, RC, _row, 0)
    pltpu.sync_copy(rows0, out_hbm.at[pl.ds(base, RC)])


def _sc_combine(ys, p0, p1):
    k = pl.kernel(
        _combine_body,
        out_type=jax.ShapeDtypeStruct((T, H), jnp.float32),
        mesh=_sc_mesh(),
        scratch_types=[
            pltpu.VMEM((RC,), jnp.int32),
            pltpu.VMEM((RC,), jnp.int32),
            pltpu.VMEM((RC, H), jnp.float32),
            pltpu.VMEM((RC, H), jnp.float32),
            pltpu.SemaphoreType.DMA,
        ],
    )
    return k(p0, p1, ys)


@jax.jit
def kernel(hidden_states, router_w, w1, b1, w2, b2):
    x = hidden_states.reshape(T, H)
    st, sw, pos, te, aux = _router(x, router_w)
    xs = _sc_gather(x, st.reshape(S))
    ys = _ffn(xs, w1, b1, w2, b2, sw, te.reshape(NT))
    out = _sc_combine(ys, pos[:, 0], pos[:, 1])
    return (out.reshape(B, T, H), aux[0, 0].astype(jnp.float32),
            jnp.float32(0.0))


# speedup vs baseline: 1.1229x; 1.1229x over previous
import functools

import jax
import jax.numpy as jnp
from jax import lax
from jax.experimental import pallas as pl
from jax.experimental.pallas import tpu as pltpu
from jax.experimental.pallas import tpu_sc as plsc

B, T, H, E, F, K = 1, 2048, 768, 8, 3072, 2
P = T * K
TM = 128
NT = P // TM + E
S = NT * TM

NC, NS = 2, 16
NW = NC * NS
RG = S // NW
RC = T // NW

_NEG = -1e30


def _router_body(x_ref, rw_ref, st_ref, sw_ref, pos_ref, te_ref, aux_ref):
    x = x_ref[...]
    rw = rw_ref[...]
    logits = lax.dot_general(x, rw, (((1,), (1,)), ((), ())),
                             preferred_element_type=jnp.float32)
    ids8 = lax.broadcasted_iota(jnp.int32, (T, E), 1)
    v0 = jnp.max(logits, axis=1, keepdims=True)
    i0 = jnp.min(jnp.where(logits == v0, ids8, E), axis=1, keepdims=True)
    masked = jnp.where(ids8 == i0, _NEG, logits)
    v1 = jnp.max(masked, axis=1, keepdims=True)
    i1 = jnp.min(jnp.where(masked == v1, ids8, E), axis=1, keepdims=True)
    e1 = jnp.exp(v1 - v0)
    w0 = 1.0 / (1.0 + e1)
    w1 = 1.0 - w0

    oh0 = (ids8 == i0).astype(jnp.float32)
    oh1 = (ids8 == i1).astype(jnp.float32)
    fullcnt = oh0 + oh1

    ri = lax.broadcasted_iota(jnp.int32, (TM, TM), 0)
    ci = lax.broadcasted_iota(jnp.int32, (TM, TM), 1)
    lincl = (ri >= ci).astype(jnp.float32)
    nb = T // TM
    acc = jnp.zeros((1, E), jnp.float32)
    cparts = []
    for b in range(nb):
        blk = fullcnt[b * TM:(b + 1) * TM, :]
        incl = lax.dot(lincl, blk, preferred_element_type=jnp.float32)
        cparts.append(incl - blk + acc)
        acc = acc + incl[TM - 1:TM, :]
    C = jnp.concatenate(cparts, axis=0)
    counts = acc

    tiles = jnp.floor((counts + (TM - 1)) * (1.0 / TM))
    er = lax.broadcasted_iota(jnp.int32, (E, E), 0)
    ec = lax.broadcasted_iota(jnp.int32, (E, E), 1)
    uexcl = (er < ec).astype(jnp.float32)
    ptile_excl = lax.dot(tiles, uexcl, preferred_element_type=jnp.float32)
    ptile_end = ptile_excl + tiles
    pstart = ptile_excl * TM

    pos0 = jnp.sum(oh0 * (pstart + C), axis=1, keepdims=True)
    pos1 = jnp.sum(oh1 * (pstart + C + oh0), axis=1, keepdims=True)
    pos_ref[...] = jnp.concatenate([pos0, pos1], axis=1).astype(jnp.int32)

    ti = lax.broadcasted_iota(jnp.int32, (NT, E), 0).astype(jnp.float32)
    te = jnp.sum((ptile_end <= ti).astype(jnp.int32), axis=1, keepdims=True)
    te_ref[...] = jnp.minimum(te, E - 1)

    tv = lax.broadcasted_iota(jnp.int32, (T, 1), 0).astype(jnp.float32)
    srow0 = lax.broadcasted_iota(jnp.int32, (1, TM), 1).astype(jnp.float32)
    for s in range(NT):
        srow = srow0 + (s * TM)
        eq0 = pos0 == srow
        eq1 = pos1 == srow
        tok = (jnp.sum(jnp.where(eq0, tv, 0.0), axis=0, keepdims=True)
               + jnp.sum(jnp.where(eq1, tv, 0.0), axis=0, keepdims=True))
        wgt = (jnp.sum(jnp.where(eq0, w0, 0.0), axis=0, keepdims=True)
               + jnp.sum(jnp.where(eq1, w1, 0.0), axis=0, keepdims=True))
        st_ref[s:s + 1, :] = tok.astype(jnp.int32)
        sw_ref[s:s + 1, :] = wgt

    p = jnp.exp(logits - v0)
    probs = p / jnp.sum(p, axis=1, keepdims=True)
    Pm = jnp.sum(probs, axis=0, keepdims=True) * (1.0 / T)
    f = counts * (1.0 / P)
    aux_ref[0, 0] = E * jnp.sum(f * Pm)


def _router(x, router_w):
    return pl.pallas_call(
        _router_body,
        out_shape=(
            jax.ShapeDtypeStruct((NT, TM), jnp.int32),
            jax.ShapeDtypeStruct((NT, TM), jnp.float32),
            jax.ShapeDtypeStruct((T, K), jnp.int32),
            jax.ShapeDtypeStruct((NT, 1), jnp.int32),
            jax.ShapeDtypeStruct((1, 1), jnp.float32),
        ),
        in_specs=[
            pl.BlockSpec((T, H), lambda: (0, 0)),
            pl.BlockSpec((E, H), lambda: (0, 0)),
        ],
        out_specs=(
            pl.BlockSpec((NT, TM), lambda: (0, 0)),
            pl.BlockSpec((NT, TM), lambda: (0, 0)),
            pl.BlockSpec((T, K), lambda: (0, 0)),
            pl.BlockSpec((NT, 1), lambda: (0, 0)),
            pl.BlockSpec(memory_space=pltpu.SMEM),
        ),
    )(x, router_w)


def _sc_mesh():
    return plsc.VectorSubcoreMesh(core_axis_name="c", subcore_axis_name="s",
                                  num_cores=NC, num_subcores=NS)


_CH = 80


def _gelu(x):
    return x * 0.5 * (1.0 + lax.erf(x * 0.7071067811865476))


def _ffn_body(te_ref, x_ref, st_ref, w1_ref, b1_ref, w2_ref, b2_ref, sw_ref,
              out_ref, w1s, w2s):
    i = pl.program_id(0)
    e = te_ref[i]
    eprev = te_ref[jnp.maximum(i - 1, 0)]

    @pl.when((i == 0) | (e != eprev))
    def _cast_weights():
        w1s[...] = w1_ref[0].astype(jnp.bfloat16)
        w2s[...] = w2_ref[0].astype(jnp.bfloat16)

    ri = lax.broadcasted_iota(jnp.int32, (TM, TM), 0)
    ci = lax.broadcasted_iota(jnp.int32, (TM, TM), 1)
    stcol = jnp.sum(jnp.where(ri == ci, st_ref[0], 0), axis=1,
                    keepdims=True)
    tok = lax.broadcasted_iota(jnp.int32, (TM, T), 1)
    sel = (tok == stcol).astype(jnp.bfloat16)
    xb = lax.dot(sel, x_ref[...],
                 preferred_element_type=jnp.float32).astype(jnp.bfloat16)
    h = lax.dot(xb, w1s[...], preferred_element_type=jnp.float32)
    h = _gelu(h + b1_ref[0])
    y = lax.dot(h.astype(jnp.bfloat16), w2s[...],
                preferred_element_type=jnp.float32) + b2_ref[0]
    wcol = jnp.sum(jnp.where(ri == ci, sw_ref[0], 0.0), axis=1,
                   keepdims=True)
    out_ref[...] = wcol * y


def _ffn(xb, st, w1, b1, w2, b2, sw, te):
    grid_spec = pltpu.PrefetchScalarGridSpec(
        num_scalar_prefetch=1,
        grid=(NT,),
        in_specs=[
            pl.BlockSpec((T, H), lambda i, te: (0, 0)),
            pl.BlockSpec((1, 1, TM), lambda i, te: (i, 0, 0)),
            pl.BlockSpec((1, H, F), lambda i, te: (te[i], 0, 0)),
            pl.BlockSpec((1, 1, F), lambda i, te: (te[i], 0, 0)),
            pl.BlockSpec((1, F, H), lambda i, te: (te[i], 0, 0)),
            pl.BlockSpec((1, 1, H), lambda i, te: (te[i], 0, 0)),
            pl.BlockSpec((1, 1, TM), lambda i, te: (i, 0, 0)),
        ],
        out_specs=pl.BlockSpec((TM, H), lambda i, te: (i, 0)),
        scratch_shapes=[
            pltpu.VMEM((H, F), jnp.bfloat16),
            pltpu.VMEM((F, H), jnp.bfloat16),
        ],
    )
    return pl.pallas_call(
        _ffn_body,
        grid_spec=grid_spec,
        out_shape=jax.ShapeDtypeStruct((S, H), jnp.float32),
    )(te, xb, st.reshape(NT, 1, TM), w1, b1.reshape(E, 1, F), w2,
      b2.reshape(E, 1, H), sw.reshape(NT, 1, TM))


def _combine_body(p0_hbm, p1_hbm, ys_hbm, out_hbm, idx0, idx1, rows0, rows1,
                  sem):
    wid = lax.axis_index("s") * NC + lax.axis_index("c")
    base = wid * RC
    pltpu.sync_copy(p0_hbm.at[pl.ds(base, RC)], idx0)
    pltpu.sync_copy(p1_hbm.at[pl.ds(base, RC)], idx1)
    c0 = pltpu.async_copy(ys_hbm.at[idx0], rows0, sem)
    c1 = pltpu.async_copy(ys_hbm.at[idx1], rows1, sem)
    c0.wait()
    c1.wait()

    def _row(j, _):
        def _chunk(i, _):
            sl = pl.ds(i * 16, 16)
            rows0[j, sl] = rows0[j, sl] + rows1[j, sl]
            return 0
        return lax.fori_loop(0, H // 16, _chunk, 0, unroll=4)

    lax.fori_loop(0, RC, _row, 0)
    pltpu.sync_copy(rows0, out_hbm.at[pl.ds(base, RC)])


def _sc_combine(ys, p0, p1):
    k = pl.kernel(
        _combine_body,
        out_type=jax.ShapeDtypeStruct((T, H), jnp.float32),
        mesh=_sc_mesh(),
        scratch_types=[
            pltpu.VMEM((RC,), jnp.int32),
            pltpu.VMEM((RC,), jnp.int32),
            pltpu.VMEM((RC, H), jnp.float32),
            pltpu.VMEM((RC, H), jnp.float32),
            pltpu.SemaphoreType.DMA,
        ],
    )
    return k(p0, p1, ys)


@jax.jit
def kernel(hidden_states, router_w, w1, b1, w2, b2):
    x = hidden_states.reshape(T, H)
    st, sw, pos, te, aux = _router(x, router_w)
    ys = _ffn(x.astype(jnp.bfloat16), st, w1, b1, w2, b2, sw,
              te.reshape(NT))
    out = _sc_combine(ys, pos[:, 0], pos[:, 1])
    return (out.reshape(B, T, H), aux[0, 0].astype(jnp.float32),
            jnp.float32(0.0))
